# baseline (speedup 1.0000x reference)
import jax
import jax.numpy as jnp
from jax.experimental import pallas as pl
from jax.experimental.pallas import tpu as pltpu

_B, _N, _F, _E = 8, 4096, 128, 65536
_C = 256
_NT = 256


def _sgcm_kernel(T_ref, x_ref, nodes_ref, edges_ref, w_ref,
                 W1s_ref, W1n_ref, b1_ref, W2s_ref, W2n_ref, b2_ref,
                 out_ref, h_scr, agg_scr):
    b = pl.program_id(0)
    Tb = T_ref[b]
    ntiles = _N // _NT
    nchunks = _E // _C

    x_row = x_ref[0]

    def init_tile(nt, carry):
        riota = jax.lax.broadcasted_iota(jnp.int32, (_NT, 1), 0) + nt * _NT
        tile = nodes_ref[0, pl.ds(nt * _NT, _NT), :]
        h_scr[pl.ds(nt * _NT, _NT), :] = jnp.where(riota == Tb, x_row, tile)
        agg_scr[pl.ds(nt * _NT, _NT), :] = jnp.zeros((_NT, _F), jnp.float32)
        return carry

    jax.lax.fori_loop(0, ntiles, init_tile, 0)

    def edge_chunk(ci):
        src = edges_ref[0, 0, pl.ds(ci * _C, _C)].reshape(1, _C)
        dst = edges_ref[0, 1, pl.ds(ci * _C, _C)].reshape(1, _C)
        wch = w_ref[0, 0, pl.ds(ci * _C, _C)].reshape(1, _C)
        return src, dst, wch

    def gather_msgs(src):
        def gat(nt, acc):
            niota = jax.lax.broadcasted_iota(jnp.int32, (_NT, _C), 0) + nt * _NT
            oh = (niota == src).astype(jnp.float32)
            ht = h_scr[pl.ds(nt * _NT, _NT), :]
            return acc + jax.lax.dot_general(
                oh, ht, (((0,), (0,)), ((), ())),
                preferred_element_type=jnp.float32)
        return jax.lax.fori_loop(0, ntiles, gat,
                                 jnp.zeros((_C, _F), jnp.float32))

    def chunk_l1(ci, carry):
        src, dst, wch = edge_chunk(ci)
        msgs = gather_msgs(src)

        def scat(nt, c2):
            niota = jax.lax.broadcasted_iota(jnp.int32, (_NT, _C), 0) + nt * _NT
            ohw = jnp.where(niota == dst, wch, 0.0)
            blk = agg_scr[pl.ds(nt * _NT, _NT), :]
            agg_scr[pl.ds(nt * _NT, _NT), :] = blk + jnp.dot(
                ohw, msgs, preferred_element_type=jnp.float32)
            return c2

        jax.lax.fori_loop(0, ntiles, scat, 0)
        return carry

    jax.lax.fori_loop(0, nchunks, chunk_l1, 0)

    W1s = W1s_ref[...]
    W1n = W1n_ref[...]
    b1v = b1_ref[...]

    def comb(nt, carry):
        h0 = h_scr[pl.ds(nt * _NT, _NT), :]
        ag = agg_scr[pl.ds(nt * _NT, _NT), :]
        h1 = jnp.tanh(
            jnp.dot(h0, W1s, preferred_element_type=jnp.float32)
            + jnp.dot(ag, W1n, preferred_element_type=jnp.float32) + b1v)
        h_scr[pl.ds(nt * _NT, _NT), :] = h1
        return carry

    jax.lax.fori_loop(0, ntiles, comb, 0)

    def chunk_l2(ci, acc):
        src, dst, wch = edge_chunk(ci)
        msgs = gather_msgs(src)
        mask = jnp.where(dst == Tb, wch, 0.0)
        return acc + jnp.dot(mask, msgs, preferred_element_type=jnp.float32)

    agg2 = jax.lax.fori_loop(0, nchunks, chunk_l2,
                             jnp.zeros((1, _F), jnp.float32))

    def hrow(nt, acc):
        riota = jax.lax.broadcasted_iota(jnp.int32, (_NT, 1), 0) + nt * _NT
        ht = h_scr[pl.ds(nt * _NT, _NT), :]
        return acc + jnp.sum(jnp.where(riota == Tb, ht, 0.0),
                             axis=0, keepdims=True)

    h1row = jax.lax.fori_loop(0, ntiles, hrow,
                              jnp.zeros((1, _F), jnp.float32))

    out_ref[0] = jnp.tanh(
        jnp.dot(h1row, W2s_ref[...], preferred_element_type=jnp.float32)
        + jnp.dot(agg2, W2n_ref[...], preferred_element_type=jnp.float32)
        + b2_ref[...])


@jax.jit
def kernel(x, taus, nodes, edges, weights, T, W1s, W1n, b1, W2s, W2n, b2):
    del taus
    b1r = b1.reshape(1, _F)
    b2r = b2.reshape(1, _F)

    grid_spec = pltpu.PrefetchScalarGridSpec(
        num_scalar_prefetch=1,
        grid=(_B,),
        in_specs=[
            pl.BlockSpec((1, 1, _F), lambda b, Tr: (b, 0, 0)),
            pl.BlockSpec((1, _N, _F), lambda b, Tr: (b, 0, 0)),
            pl.BlockSpec((1, 2, _E), lambda b, Tr: (b, 0, 0)),
            pl.BlockSpec((1, 1, _E), lambda b, Tr: (b, 0, 0)),
            pl.BlockSpec((_F, _F), lambda b, Tr: (0, 0)),
            pl.BlockSpec((_F, _F), lambda b, Tr: (0, 0)),
            pl.BlockSpec((1, _F), lambda b, Tr: (0, 0)),
            pl.BlockSpec((_F, _F), lambda b, Tr: (0, 0)),
            pl.BlockSpec((_F, _F), lambda b, Tr: (0, 0)),
            pl.BlockSpec((1, _F), lambda b, Tr: (0, 0)),
        ],
        out_specs=pl.BlockSpec((1, 1, _F), lambda b, Tr: (b, 0, 0)),
        scratch_shapes=[
            pltpu.VMEM((_N, _F), jnp.float32),
            pltpu.VMEM((_N, _F), jnp.float32),
        ],
    )

    out = pl.pallas_call(
        _sgcm_kernel,
        grid_spec=grid_spec,
        out_shape=jax.ShapeDtypeStruct((_B, 1, _F), jnp.float32),
    )(T, x, nodes, edges, weights, W1s, W1n, b1r, W2s, W2n, b2r)
    return out.reshape(_B, _F)
